# Initial kernel scaffold; baseline (speedup 1.0000x reference)
import functools

import jax
import jax.numpy as jnp
from jax import lax
from jax.experimental import pallas as pl
from jax.experimental.pallas import tpu as pltpu
from jax.experimental.pallas import tpu_sc as plsc

_F32 = jnp.float32
_NC = 2
_NS = 16
_NW = _NC * _NS
_CHUNK = 128


def _dot(a, b):
    return lax.dot_general(a, b, (((1,), (0,)), ((), ())),
                           preferred_element_type=_F32,
                           precision=lax.Precision.HIGHEST)


def _enc_body(x_ref, w1_ref, b1_ref, w2_ref, b2_ref, wr_ref, h_ref, hr_ref):
    h1 = jnp.maximum(_dot(x_ref[...], w1_ref[...]) + b1_ref[...], 0.0)
    h = jnp.maximum(_dot(h1, w2_ref[...]) + b2_ref[...], 0.0)
    h_ref[...] = h
    hr_ref[...] = _dot(h, wr_ref[...])


def _encode(x, w1t, b1, w2t, b2, wrt, blk):
    n, d_in = x.shape
    d_hid = w2t.shape[1]
    full = lambda a: pl.BlockSpec(a.shape, lambda i: (0,) * a.ndim)
    return pl.pallas_call(
        _enc_body,
        grid=(n // blk,),
        in_specs=[pl.BlockSpec((blk, d_in), lambda i: (i, 0)),
                  full(w1t), full(b1), full(w2t), full(b2), full(wrt)],
        out_specs=[pl.BlockSpec((blk, d_hid), lambda i: (i, 0)),
                   pl.BlockSpec((blk, d_hid), lambda i: (i, 0))],
        out_shape=[jax.ShapeDtypeStruct((n, d_hid), _F32),
                   jax.ShapeDtypeStruct((n, d_hid), _F32)],
    )(x, w1t, b1, w2t, b2, wrt)


def _comb_mid_body(s0_ref, s1_ref, c0_ref, c1_ref, hr_ref, wl_ref, bl_ref,
                   wrn_ref, h_ref, hrn_ref):
    cnt = jnp.maximum(c0_ref[...][:, :1] + c1_ref[...][:, :1], 1.0)
    mean = (s0_ref[...] + s1_ref[...]) / cnt
    h = jnp.maximum(_dot(mean, wl_ref[...]) + bl_ref[...] + hr_ref[...], 0.0)
    h_ref[...] = h
    hrn_ref[...] = _dot(h, wrn_ref[...])


def _combine_mid(s0, s1, c0, c1, hr, wlt, bl, wrnt, blk):
    n, d = s0.shape
    full = lambda a: pl.BlockSpec(a.shape, lambda i: (0,) * a.ndim)
    row = lambda w: pl.BlockSpec((blk, w), lambda i: (i, 0))
    return pl.pallas_call(
        _comb_mid_body,
        grid=(n // blk,),
        in_specs=[row(d), row(d), row(16), row(16), row(d),
                  full(wlt), full(bl), full(wrnt)],
        out_specs=[row(d), row(d)],
        out_shape=[jax.ShapeDtypeStruct((n, d), _F32),
                   jax.ShapeDtypeStruct((n, d), _F32)],
    )(s0, s1, c0, c1, hr, wlt, bl, wrnt)


def _comb_dec_body(s0_ref, s1_ref, c0_ref, c1_ref, hr_ref, wl_ref, bl_ref,
                   d1_ref, db1_ref, d2_ref, db2_ref, o_ref):
    cnt = jnp.maximum(c0_ref[...][:, :1] + c1_ref[...][:, :1], 1.0)
    mean = (s0_ref[...] + s1_ref[...]) / cnt
    h = jnp.maximum(_dot(mean, wl_ref[...]) + bl_ref[...] + hr_ref[...], 0.0)
    d = jnp.maximum(_dot(h, d1_ref[...]) + db1_ref[...], 0.0)
    o_ref[...] = _dot(d, d2_ref[...]) + db2_ref[...]


def _combine_dec(s0, s1, c0, c1, hr, wlt, bl, d1t, db1, d2t, db2, blk):
    n, d = s0.shape
    d_out = d2t.shape[1]
    full = lambda a: pl.BlockSpec(a.shape, lambda i: (0,) * a.ndim)
    row = lambda w: pl.BlockSpec((blk, w), lambda i: (i, 0))
    return pl.pallas_call(
        _comb_dec_body,
        grid=(n // blk,),
        in_specs=[row(d), row(d), row(16), row(16), row(d),
                  full(wlt), full(bl), full(d1t), full(db1), full(d2t),
                  full(db2)],
        out_specs=pl.BlockSpec((blk, d_out), lambda i: (i, 0)),
        out_shape=jax.ShapeDtypeStruct((n, d_out), _F32),
    )(s0, s1, c0, c1, hr, wlt, bl, d1t, db1, d2t, db2)


def _agg(table, src_p, dst_p, zsum, zcnt, ones, n_pad, chunks_pw,
         with_counts):
    d = table.shape[1]
    rps = n_pad // _NS
    mesh = plsc.VectorSubcoreMesh(core_axis_name="c", subcore_axis_name="s")

    out_type = [jax.ShapeDtypeStruct((_NC, n_pad, d), _F32)]
    scratch = [
        pltpu.VMEM((chunks_pw, _CHUNK), jnp.int32),
        pltpu.VMEM((chunks_pw, _CHUNK), jnp.int32),
        pltpu.VMEM((_CHUNK, d), _F32),
        pltpu.VMEM_SHARED((n_pad, d), _F32),
        pltpu.SemaphoreType.DMA,
    ]
    if with_counts:
        out_type.append(jax.ShapeDtypeStruct((_NC, n_pad, 16), _F32))
        scratch += [
            pltpu.VMEM((_CHUNK, 16), _F32),
            pltpu.VMEM_SHARED((n_pad, 16), _F32),
        ]

    def body(refs):
        if with_counts:
            (tb, src_h, dst_h, zs_h, zc_h, on_h, osum, ocnt,
             src_v, dst_v, rows_v, acc, sem, ones_v, cacc) = refs
        else:
            (tb, src_h, dst_h, zs_h, osum,
             src_v, dst_v, rows_v, acc, sem) = refs
            ocnt = zc_h = on_h = ones_v = cacc = None
        c = lax.axis_index("c")
        s = lax.axis_index("s")
        wid = c * _NS + s
        stripe = pl.ds(s * rps, rps)
        pltpu.sync_copy(zs_h, acc.at[stripe])
        pltpu.sync_copy(src_h.at[wid], src_v)
        pltpu.sync_copy(dst_h.at[wid], dst_v)
        if with_counts:
            pltpu.sync_copy(zc_h, cacc.at[stripe])
            pltpu.sync_copy(on_h, ones_v)
        plsc.subcore_barrier()

        @pl.loop(0, chunks_pw)
        def _(j):
            pltpu.async_copy(tb.at[src_v.at[j]], rows_v, sem).wait()
            pltpu.sync_copy(rows_v, acc.at[dst_v.at[j]], add=True)
            if with_counts:
                pltpu.sync_copy(ones_v, cacc.at[dst_v.at[j]], add=True)

        plsc.subcore_barrier()
        pltpu.sync_copy(acc.at[stripe], osum.at[c].at[stripe])
        if with_counts:
            pltpu.sync_copy(cacc.at[stripe], ocnt.at[c].at[stripe])

    def kern(*refs):
        body(refs)

    k = pl.kernel(kern, out_type=tuple(out_type), mesh=mesh,
                  scratch_types=scratch)
    if with_counts:
        return k(table, src_p, dst_p, zsum, zcnt, ones)
    return k(table, src_p, dst_p, zsum)


def kernel(x, edge_index, enc_W1, enc_b1, enc_W2, enc_b2,
           s1_Wl, s1_bl, s1_Wr, s2_Wl, s2_bl, s2_Wr,
           dec_W1, dec_b1, dec_W2, dec_b2):
    n, _ = x.shape
    e = edge_index.shape[1]

    chunks_pw = -(-e // (_NW * _CHUNK))
    e_pad = chunks_pw * _NW * _CHUNK
    src = edge_index[0].astype(jnp.int32)
    dst = edge_index[1].astype(jnp.int32)
    pad = e_pad - e
    src_p = jnp.concatenate([src, jnp.zeros((pad,), jnp.int32)])
    dst_p = jnp.concatenate([dst, jnp.full((pad,), n, jnp.int32)])
    src_p = src_p.reshape(_NW, chunks_pw, _CHUNK)
    dst_p = dst_p.reshape(_NW, chunks_pw, _CHUNK)

    n_pad = ((n + _NS) // _NS) * _NS
    zsum = jnp.zeros((n_pad // _NS, 128), _F32)
    zcnt = jnp.zeros((n_pad // _NS, 16), _F32)
    ones = jnp.ones((_CHUNK, 16), _F32)

    blk = 1000
    w1t, w2t = enc_W1.T, enc_W2.T
    wl1t, wr1t = s1_Wl.T, s1_Wr.T
    wl2t, wr2t = s2_Wl.T, s2_Wr.T
    d1t, d2t = dec_W1.T, dec_W2.T

    h1, h1r = _encode(x, w1t, enc_b1[None], w2t, enc_b2[None], wr1t, blk)
    sums1, cnts1 = _agg(h1, src_p, dst_p, zsum, zcnt, ones, n_pad,
                        chunks_pw, with_counts=True)
    c0 = cnts1[0, :n]
    c1 = cnts1[1, :n]
    h2, h2r = _combine_mid(sums1[0, :n], sums1[1, :n], c0, c1, h1r,
                           wl1t, s1_bl[None], wr2t, blk)
    sums2 = _agg(h2, src_p, dst_p, zsum, None, None, n_pad,
                 chunks_pw, with_counts=False)
    if isinstance(sums2, (tuple, list)):
        sums2 = sums2[0]
    out = _combine_dec(sums2[0, :n], sums2[1, :n], c0, c1, h2r,
                       wl2t, s2_bl[None], d1t, dec_b1[None], d2t,
                       dec_b2[None], blk)
    return out


# speedup vs baseline: 3.0051x; 3.0051x over previous
import jax
import jax.numpy as jnp
from jax import lax
from jax.experimental import pallas as pl
from jax.experimental.pallas import tpu as pltpu
from jax.experimental.pallas import tpu_sc as plsc

_F32 = jnp.float32
_NC = 2
_NS = 16
_NW = _NC * _NS
_NL = 16
_CHUNK = 128
_NP = 8
_PW = 16


def _dot(a, b):
    return lax.dot_general(a, b, (((1,), (0,)), ((), ())),
                           preferred_element_type=_F32,
                           precision=lax.Precision.HIGHEST)


def _enc_body(x_ref, w1_ref, b1_ref, w2_ref, b2_ref, wr_ref, h8_ref, hr_ref):
    h1 = jnp.maximum(_dot(x_ref[...], w1_ref[...]) + b1_ref[...], 0.0)
    h = jnp.maximum(_dot(h1, w2_ref[...]) + b2_ref[...], 0.0)
    for p in range(_NP):
        h8_ref[p] = h[:, p * _PW:(p + 1) * _PW]
    hr_ref[...] = _dot(h, wr_ref[...])


def _encode(x, w1t, b1, w2t, b2, wrt, blk):
    n, d_in = x.shape
    d_hid = w2t.shape[1]
    full = lambda a: pl.BlockSpec(a.shape, lambda i: (0,) * a.ndim)
    return pl.pallas_call(
        _enc_body,
        grid=(n // blk,),
        in_specs=[pl.BlockSpec((blk, d_in), lambda i: (i, 0)),
                  full(w1t), full(b1), full(w2t), full(b2), full(wrt)],
        out_specs=[pl.BlockSpec((_NP, blk, _PW), lambda i: (0, i, 0)),
                   pl.BlockSpec((blk, d_hid), lambda i: (i, 0))],
        out_shape=[jax.ShapeDtypeStruct((_NP, n, _PW), _F32),
                   jax.ShapeDtypeStruct((n, d_hid), _F32)],
    )(x, w1t, b1, w2t, b2, wrt)


def _mean_linl(s_ref, c_ref, wl_ref):
    inv = 1.0 / jnp.maximum(c_ref[...], 1.0)
    s = s_ref[...]
    acc = None
    for p in range(_NP):
        mean_p = (s[0, p] + s[1, p]) * inv
        part = _dot(mean_p, wl_ref[p * _PW:(p + 1) * _PW, :])
        acc = part if acc is None else acc + part
    return acc


def _comb_mid_body(s_ref, c_ref, hr_ref, wl_ref, bl_ref, wrn_ref,
                   h8_ref, hrn_ref):
    h = jnp.maximum(_mean_linl(s_ref, c_ref, wl_ref) + bl_ref[...]
                    + hr_ref[...], 0.0)
    for p in range(_NP):
        h8_ref[p] = h[:, p * _PW:(p + 1) * _PW]
    hrn_ref[...] = _dot(h, wrn_ref[...])


def _combine_mid(sums, cnt, hr, wlt, bl, wrnt, blk):
    n, d = hr.shape
    full = lambda a: pl.BlockSpec(a.shape, lambda i: (0,) * a.ndim)
    row = lambda w: pl.BlockSpec((blk, w), lambda i: (i, 0))
    sspec = pl.BlockSpec((_NC, _NP, blk, _PW), lambda i: (0, 0, i, 0))
    cspec = pl.BlockSpec((blk, 1), lambda i: (i, 0))
    return pl.pallas_call(
        _comb_mid_body,
        grid=(n // blk,),
        in_specs=[sspec, cspec, row(d), full(wlt), full(bl), full(wrnt)],
        out_specs=[pl.BlockSpec((_NP, blk, _PW), lambda i: (0, i, 0)),
                   row(d)],
        out_shape=[jax.ShapeDtypeStruct((_NP, n, _PW), _F32),
                   jax.ShapeDtypeStruct((n, d), _F32)],
    )(sums, cnt, hr, wlt, bl, wrnt)


def _comb_dec_body(s_ref, c_ref, hr_ref, wl_ref, bl_ref, d1_ref, db1_ref,
                   d2_ref, db2_ref, o_ref):
    h = jnp.maximum(_mean_linl(s_ref, c_ref, wl_ref) + bl_ref[...]
                    + hr_ref[...], 0.0)
    d = jnp.maximum(_dot(h, d1_ref[...]) + db1_ref[...], 0.0)
    o_ref[...] = _dot(d, d2_ref[...]) + db2_ref[...]


def _combine_dec(sums, cnt, hr, wlt, bl, d1t, db1, d2t, db2, blk):
    n, d = hr.shape
    d_out = d2t.shape[1]
    full = lambda a: pl.BlockSpec(a.shape, lambda i: (0,) * a.ndim)
    row = lambda w: pl.BlockSpec((blk, w), lambda i: (i, 0))
    sspec = pl.BlockSpec((_NC, _NP, blk, _PW), lambda i: (0, 0, i, 0))
    cspec = pl.BlockSpec((blk, 1), lambda i: (i, 0))
    return pl.pallas_call(
        _comb_dec_body,
        grid=(n // blk,),
        in_specs=[sspec, cspec, row(d), full(wlt), full(bl), full(d1t),
                  full(db1), full(d2t), full(db2)],
        out_specs=pl.BlockSpec((blk, d_out), lambda i: (i, 0)),
        out_shape=jax.ShapeDtypeStruct((n, d_out), _F32),
    )(sums, cnt, hr, wlt, bl, d1t, db1, d2t, db2)


def _hist_reduce_body(h_ref, o_ref):
    o_ref[...] = jnp.sum(h_ref[...], axis=0)[:, None]


def _hist_reduce(hist):
    nw, n_pad = hist.shape
    return pl.pallas_call(
        _hist_reduce_body,
        grid=(1,),
        in_specs=[pl.BlockSpec((nw, n_pad), lambda i: (0, 0))],
        out_specs=pl.BlockSpec((n_pad, 1), lambda i: (0, 0)),
        out_shape=jax.ShapeDtypeStruct((n_pad, 1), _F32),
    )(hist)


def _make_agg(n_pad, chunks_pw):
    rps = n_pad // _NS
    mesh = plsc.VectorSubcoreMesh(core_axis_name="c", subcore_axis_name="s")
    cparams = pltpu.CompilerParams(use_tc_tiling_on_sc=False,
                                   needs_layout_passes=False)

    @pl.kernel(
        out_type=(jax.ShapeDtypeStruct((_NC, _NP, n_pad, _PW), _F32),
                  jax.ShapeDtypeStruct((_NW, n_pad), _F32)),
        mesh=mesh,
        scratch_types=[
            pltpu.VMEM((chunks_pw, _CHUNK), jnp.int32),
            pltpu.VMEM((chunks_pw, _CHUNK), jnp.int32),
            pltpu.VMEM((_CHUNK, _PW), _F32),
            pltpu.VMEM((n_pad,), _F32),
            pltpu.VMEM_SHARED((n_pad, _PW), _F32),
            pltpu.SemaphoreType.DMA,
        ],
        compiler_params=cparams,
    )
    def k(tb, src_h, dst_h, zs_h, osum, ohist,
          src_v, dst_v, rows_v, hist_v, acc, sem):
        c = lax.axis_index("c")
        s = lax.axis_index("s")
        wid = c * _NS + s
        stripe = pl.ds(s * rps, rps)
        pltpu.sync_copy(dst_h.at[wid], dst_v)

        one_v = jnp.ones((_NL,), _F32)
        zero_v = jnp.zeros((_NL,), _F32)

        @pl.loop(0, n_pad // _NL)
        def _(i):
            hist_v[pl.ds(i * _NL, _NL)] = zero_v

        @pl.loop(0, _NP)
        def _(p):
            pltpu.sync_copy(zs_h, acc.at[stripe])
            pltpu.sync_copy(src_h.at[p].at[wid], src_v)
            plsc.subcore_barrier()

            @pl.loop(0, chunks_pw)
            def _(j):
                pltpu.async_copy(tb.at[src_v.at[j]], rows_v, sem).wait()
                pltpu.sync_copy(rows_v, acc.at[dst_v.at[j]], add=True)

                @pl.when(p == 0)
                def _():
                    for kk in range(_CHUNK // _NL):
                        idx = dst_v[j, pl.ds(kk * _NL, _NL)]
                        plsc.addupdate_scatter(hist_v, [idx], one_v)

            plsc.subcore_barrier()
            pltpu.sync_copy(acc.at[stripe], osum.at[c].at[p].at[stripe])

        pltpu.sync_copy(hist_v, ohist.at[wid])

    return k


def kernel(x, edge_index, enc_W1, enc_b1, enc_W2, enc_b2,
           s1_Wl, s1_bl, s1_Wr, s2_Wl, s2_bl, s2_Wr,
           dec_W1, dec_b1, dec_W2, dec_b2):
    n, _ = x.shape
    e = edge_index.shape[1]

    chunks_pw = -(-e // (_NW * _CHUNK))
    e_pad = chunks_pw * _NW * _CHUNK
    src = edge_index[0].astype(jnp.int32)
    dst = edge_index[1].astype(jnp.int32)
    pad = e_pad - e
    src_p = jnp.concatenate([src, jnp.zeros((pad,), jnp.int32)])
    dst_p = jnp.concatenate([dst, jnp.full((pad,), n, jnp.int32)])
    src_p = src_p.reshape(_NW, chunks_pw, _CHUNK)
    dst_p = dst_p.reshape(_NW, chunks_pw, _CHUNK)
    src_p8 = (src_p[None] +
              (jnp.arange(_NP, dtype=jnp.int32) * n)[:, None, None, None])

    n_pad = ((n + _NS * 8) // (_NS * 8)) * (_NS * 8)
    zsum = jnp.zeros((n_pad // _NS, _PW), _F32)

    blk = 1000
    w1t, w2t = enc_W1.T, enc_W2.T
    wr1t, wr2t = s1_Wr.T, s2_Wr.T
    wl1t, wl2t = s1_Wl.T, s2_Wl.T
    d1t, d2t = dec_W1.T, dec_W2.T

    agg = _make_agg(n_pad, chunks_pw)
    h1_8, h1r = _encode(x, w1t, enc_b1[None], w2t, enc_b2[None], wr1t, blk)
    sums1, hist = agg(h1_8.reshape(_NP * n, _PW), src_p8, dst_p, zsum)
    cnt = _hist_reduce(hist)[:n]
    h2_8, h2r = _combine_mid(sums1[:, :, :n], cnt, h1r, wl1t, s1_bl[None],
                             wr2t, blk)
    sums2, _ = agg(h2_8.reshape(_NP * n, _PW), src_p8, dst_p, zsum)
    out = _combine_dec(sums2[:, :, :n], cnt, h2r, wl2t, s2_bl[None], d1t,
                       dec_b1[None], d2t, dec_b2[None], blk)
    return out
